# Initial kernel scaffold; baseline (speedup 1.0000x reference)
import jax
import jax.numpy as jnp
from jax.experimental import pallas as pl


def kernel(x, edge_index, W1_l, W1_r, b1, W2_l, W2_r, b2, W3_l, W3_r, b3):
    raise NotImplementedError("write your pallas kernel here")


# speedup vs baseline: 4.0962x; 4.0962x over previous
import functools

import jax
import jax.numpy as jnp
from jax import lax
from jax.experimental import pallas as pl
from jax.experimental.pallas import tpu as pltpu
from jax.experimental.pallas import tpu_sc as plsc

_N = 10000
_D = 128
_E = 320000

_C = 128
_NTILE = 32
_PER_TILE = 79
_EPAD = _PER_TILE * _NTILE * _C
_NPAD = 10240
_ROWS_PER_TILE = _NPAD // 16
_CW = 16


def _sc_agg(d):
  mesh = plsc.VectorSubcoreMesh(core_axis_name="c", subcore_axis_name="s")
  out_type = jax.ShapeDtypeStruct((2 * _NPAD, d), jnp.float32)
  scratch = [
      pltpu.VMEM((_C,), jnp.int32),
      pltpu.VMEM((_C,), jnp.int32),
      pltpu.VMEM((_C, d), jnp.float32),
      pltpu.VMEM_SHARED((_NPAD, d), jnp.float32),
  ]
  n_blk = _ROWS_PER_TILE // _C

  @functools.partial(pl.kernel, out_type=out_type, mesh=mesh,
                     scratch_types=scratch)
  def k(src_hbm, dst_hbm, h_hbm, z_hbm, sum_out, src_v, dst_v, rows_v, acc_sh):
    core = lax.axis_index("c")
    sub = lax.axis_index("s")
    wid = sub * 2 + core
    zr = sub * _ROWS_PER_TILE

    pltpu.sync_copy(z_hbm, rows_v)
    for j in range(n_blk):
      pltpu.sync_copy(rows_v, acc_sh.at[pl.ds(zr + j * _C, _C)])
    plsc.subcore_barrier()

    @pl.loop(0, _PER_TILE)
    def _(i):
      off = (wid + i * _NTILE) * _C
      pltpu.sync_copy(src_hbm.at[pl.ds(off, _C)], src_v)
      pltpu.sync_copy(dst_hbm.at[pl.ds(off, _C)], dst_v)
      pltpu.sync_copy(h_hbm.at[src_v], rows_v)
      pltpu.sync_copy(rows_v, acc_sh.at[dst_v], add=True)

    plsc.subcore_barrier()
    for j in range(n_blk):
      sl = pl.ds(zr + j * _C, _C)
      ob = pl.ds(core * _NPAD + zr + j * _C, _C)
      pltpu.sync_copy(acc_sh.at[sl], rows_v)
      pltpu.sync_copy(rows_v, sum_out.at[ob])

  return k


def _sc_counts():
  mesh = plsc.VectorSubcoreMesh(core_axis_name="c", subcore_axis_name="s")
  out_type = jax.ShapeDtypeStruct((2 * _NPAD, _D), jnp.float32)
  scratch = [
      pltpu.VMEM((_C,), jnp.int32),
      pltpu.VMEM((_C, _D), jnp.float32),
      pltpu.VMEM((_C, _D), jnp.float32),
      pltpu.VMEM_SHARED((_NPAD, _D), jnp.float32),
  ]
  n_blk = _ROWS_PER_TILE // _C

  @functools.partial(pl.kernel, out_type=out_type, mesh=mesh,
                     scratch_types=scratch)
  def k(dst_hbm, ones_hbm, z_hbm, cnt_out, dst_v, ones_v, stage_v, acc_sh):
    core = lax.axis_index("c")
    sub = lax.axis_index("s")
    wid = sub * 2 + core
    zr = sub * _ROWS_PER_TILE

    pltpu.sync_copy(z_hbm, stage_v)
    for j in range(n_blk):
      pltpu.sync_copy(stage_v, acc_sh.at[pl.ds(zr + j * _C, _C)])
    pltpu.sync_copy(ones_hbm, ones_v)
    plsc.subcore_barrier()

    @pl.loop(0, _PER_TILE)
    def _(i):
      off = (wid + i * _NTILE) * _C
      pltpu.sync_copy(dst_hbm.at[pl.ds(off, _C)], dst_v)
      pltpu.sync_copy(ones_v, acc_sh.at[dst_v], add=True)

    plsc.subcore_barrier()
    for j in range(n_blk):
      pltpu.sync_copy(acc_sh.at[pl.ds(zr + j * _C, _C)], stage_v)
      pltpu.sync_copy(stage_v, cnt_out.at[pl.ds(core * _NPAD + zr + j * _C, _C)])

  return k


_R = 2000


def _dense(relu):
  def body(s0, s1, c0, c1, h, wl, wr, b, out):
    cnt = jnp.maximum(c0[:, 0:1] + c1[:, 0:1], 1.0)
    mean = (s0[...] + s1[...]) / cnt
    y = jnp.dot(mean, wl[...], precision=lax.Precision.HIGHEST,
                preferred_element_type=jnp.float32)
    y = y + jnp.dot(h[...], wr[...], precision=lax.Precision.HIGHEST,
                    preferred_element_type=jnp.float32)
    y = y + b[...]
    out[...] = jnp.maximum(y, 0.0) if relu else y

  return pl.pallas_call(
      body,
      grid=(_N // _R,),
      in_specs=[
          pl.BlockSpec((_R, _D), lambda i: (i, 0)),
          pl.BlockSpec((_R, _D), lambda i: (i, 0)),
          pl.BlockSpec((_R, _CW), lambda i: (i, 0)),
          pl.BlockSpec((_R, _CW), lambda i: (i, 0)),
          pl.BlockSpec((_R, _D), lambda i: (i, 0)),
          pl.BlockSpec((_D, _D), lambda i: (0, 0)),
          pl.BlockSpec((_D, _D), lambda i: (0, 0)),
          pl.BlockSpec((1, _D), lambda i: (0, 0)),
      ],
      out_specs=pl.BlockSpec((_R, _D), lambda i: (i, 0)),
      out_shape=jax.ShapeDtypeStruct((_N, _D), jnp.float32),
  )


def kernel(x, edge_index, W1_l, W1_r, b1, W2_l, W2_r, b2, W3_l, W3_r, b3):
  src = edge_index[0]
  dst = edge_index[1]
  pad = _EPAD - _E
  src_p = jnp.concatenate([src, jnp.zeros((pad,), jnp.int32)])
  dst_p = jnp.concatenate([dst, jnp.full((pad,), _NPAD - 1, jnp.int32)])
  z_d = jnp.zeros((_C, _D), jnp.float32)
  ones = jnp.ones((_C, _D), jnp.float32)

  cagg = _sc_counts()(dst_p, ones, z_d)
  c0, c1 = cagg[:_N, :_CW], cagg[_NPAD:_NPAD + _N, :_CW]
  s1 = _sc_agg(_D)(src_p, dst_p, x, z_d)
  h1 = _dense(True)(s1[:_N], s1[_NPAD:_NPAD + _N], c0, c1, x,
                    W1_l, W1_r, b1.reshape(1, _D))
  s2 = _sc_agg(_D)(src_p, dst_p, h1, z_d)
  h2 = _dense(True)(s2[:_N], s2[_NPAD:_NPAD + _N], c0, c1, h1,
                    W2_l, W2_r, b2.reshape(1, _D))
  s3 = _sc_agg(_D)(src_p, dst_p, h2, z_d)
  out = _dense(False)(s3[:_N], s3[_NPAD:_NPAD + _N], c0, c1, h2,
                      W3_l, W3_r, b3.reshape(1, _D))
  return out
